# Initial kernel scaffold; baseline (speedup 1.0000x reference)
import jax
import jax.numpy as jnp
from jax.experimental import pallas as pl


def kernel(cat_data, num_data, emb_tables, W1, b1, g1, be1, W2, b2, g2, be2, W3, b3):
    raise NotImplementedError("write your pallas kernel here")


# speedup vs baseline: 4.9121x; 4.9121x over previous
import functools

import jax
import jax.numpy as jnp
from jax import lax
from jax.experimental import pallas as pl
from jax.experimental.pallas import tpu as pltpu
from jax.experimental.pallas import tpu_sc as plsc

_B, _F, _V, _D = 16384, 26, 100000, 50
_EPS = 1e-5
_NC, _NS = 2, 16
_NW = _NC * _NS
_BF = _B * _F
_PER_W = _BF // _NW
_CH = 128
_NCH = _PER_W // _CH
_TW = (_F * _V * _D) // 16


def _gather_body(table_hbm, idx4_hbm, shift_hbm, out_hbm,
                 idx4_v, shift_v, win_a, win_b, out_a, out_b,
                 sem_a, sem_b, sem_oa, sem_ob):
    wid = lax.axis_index("s") * _NC + lax.axis_index("c")
    pltpu.sync_copy(idx4_hbm.at[pl.ds(wid * 4 * _NCH, 4 * _NCH)], idx4_v)
    pltpu.sync_copy(shift_hbm.at[pl.ds(wid * _NCH, _NCH)], shift_v)

    def fire(j, win, sem):
        for t in range(4):
            pltpu.async_copy(table_hbm.at[idx4_v.at[j * 4 + t]],
                             win.at[pl.ds(t * 128, 128)], sem)

    def drain(win, sem):
        for t in range(4):
            pltpu.make_async_copy(table_hbm.at[idx4_v.at[0]],
                                  win.at[pl.ds(t * 128, 128)], sem).wait()

    fire(0, win_a, sem_a)

    def body(j, _):
        even = lax.rem(j, 2) == 0

        @pl.when(jnp.logical_and(j + 1 < _NCH, even))
        def _():
            fire(j + 1, win_b, sem_b)

        @pl.when(jnp.logical_and(j + 1 < _NCH, jnp.logical_not(even)))
        def _():
            fire(j + 1, win_a, sem_a)

        def realign_store(win, out, sem_o):
            @pl.when(j >= 2)
            def _():
                pltpu.make_async_copy(out.at[pl.ds(0, _CH * _D)],
                                      out_hbm.at[pl.ds(0, _CH * _D)],
                                      sem_o).wait()
            iota = lax.iota(jnp.int32, 16)
            for q in range(8):
                sv = shift_v[j, pl.ds(q * 16, 16)]
                for t in range(16):
                    r = q * 16 + t
                    flat = sv[t] + iota
                    cols = lax.bitwise_and(flat, 15)
                    rows = lax.shift_right_logical(flat, 4) + (4 * r)
                    for u in range(4):
                        out[pl.ds(r * _D + u * 16, 16)] = plsc.load_gather(
                            win, [rows + u, cols])
            pltpu.async_copy(
                out.at[pl.ds(0, _CH * _D)],
                out_hbm.at[pl.ds((wid * _PER_W + j * _CH) * _D, _CH * _D)],
                sem_o)

        @pl.when(even)
        def _():
            drain(win_a, sem_a)
            realign_store(win_a, out_a, sem_oa)

        @pl.when(jnp.logical_not(even))
        def _():
            drain(win_b, sem_b)
            realign_store(win_b, out_b, sem_ob)

        return 0

    lax.fori_loop(0, _NCH, body, 0)
    pltpu.make_async_copy(out_a.at[pl.ds(0, _CH * _D)],
                          out_hbm.at[pl.ds(0, _CH * _D)], sem_oa).wait()
    pltpu.make_async_copy(out_b.at[pl.ds(0, _CH * _D)],
                          out_hbm.at[pl.ds(0, _CH * _D)], sem_ob).wait()


@functools.cache
def _sc_gather():
    return pl.kernel(
        _gather_body,
        out_type=jax.ShapeDtypeStruct((_BF * _D,), jnp.float32),
        mesh=plsc.VectorSubcoreMesh(
            core_axis_name="c", subcore_axis_name="s",
            num_cores=_NC, num_subcores=_NS),
        scratch_types=[
            pltpu.VMEM((4 * _NCH, _CH), jnp.int32),
            pltpu.VMEM((_NCH, _CH), jnp.int32),
            pltpu.VMEM((516, 16), jnp.float32),
            pltpu.VMEM((516, 16), jnp.float32),
            pltpu.VMEM((_CH * _D + 16,), jnp.float32),
            pltpu.VMEM((_CH * _D + 16,), jnp.float32),
            pltpu.SemaphoreType.DMA,
            pltpu.SemaphoreType.DMA,
            pltpu.SemaphoreType.DMA,
            pltpu.SemaphoreType.DMA,
        ],
        compiler_params=pltpu.CompilerParams(
            use_tc_tiling_on_sc=False, needs_layout_passes=False),
    )


_BB = 512


def _mlp1_body(x_ref, num_ref, w1a_ref, w1b_ref, b1_ref, h1_ref, stats_ref, acc_ref):
    i = pl.program_id(0)

    @pl.when(i == 0)
    def _():
        acc_ref[...] = jnp.zeros_like(acc_ref)

    h = jnp.dot(x_ref[...], w1a_ref[...], preferred_element_type=jnp.float32)
    h = h + jnp.dot(num_ref[...], w1b_ref[...], preferred_element_type=jnp.float32)
    h = jnp.maximum(h + b1_ref[...], 0.0)
    h1_ref[...] = h
    acc_ref[0:1, :] += jnp.sum(h, axis=0, keepdims=True)
    acc_ref[1:2, :] += jnp.sum(h * h, axis=0, keepdims=True)

    @pl.when(i == pl.num_programs(0) - 1)
    def _():
        stats_ref[...] = acc_ref[...]


def _mlp2_body(h1_ref, stats_ref, g1_ref, be1_ref, w2_ref, b2_ref, g2_ref,
               be2_ref, w3r_ref, b3_ref, out_ref):
    s = stats_ref[...]
    m1 = s[0:1, :] * (1.0 / _B)
    v1 = s[1:2, :] * (1.0 / _B) - m1 * m1
    h1n = (h1_ref[...] - m1) * lax.rsqrt(v1 + _EPS) * g1_ref[...] + be1_ref[...]
    h2 = jnp.dot(h1n, w2_ref[...], preferred_element_type=jnp.float32)
    h2 = jnp.maximum(h2 + b2_ref[...], 0.0)
    m2 = jnp.mean(h2, axis=0, keepdims=True)
    v2 = jnp.mean(h2 * h2, axis=0, keepdims=True) - m2 * m2
    h2n = (h2 - m2) * lax.rsqrt(v2 + _EPS) * g2_ref[...] + be2_ref[...]
    out_ref[...] = jnp.sum(h2n * w3r_ref[...], axis=1, keepdims=True) + b3_ref[...]


def _mlp1(x, nump, w1a, w1b, b1r):
    grid = (_B // _BB,)
    return pl.pallas_call(
        _mlp1_body,
        grid=grid,
        in_specs=[
            pl.BlockSpec((_BB, _F * _D), lambda i: (i, 0)),
            pl.BlockSpec((_BB, 16), lambda i: (i, 0)),
            pl.BlockSpec((_F * _D, 128), lambda i: (0, 0)),
            pl.BlockSpec((16, 128), lambda i: (0, 0)),
            pl.BlockSpec((1, 128), lambda i: (0, 0)),
        ],
        out_specs=[
            pl.BlockSpec((_BB, 128), lambda i: (i, 0)),
            pl.BlockSpec((8, 128), lambda i: (0, 0)),
        ],
        out_shape=[
            jax.ShapeDtypeStruct((_B, 128), jnp.float32),
            jax.ShapeDtypeStruct((8, 128), jnp.float32),
        ],
        scratch_shapes=[pltpu.VMEM((8, 128), jnp.float32)],
    )(x, nump, w1a, w1b, b1r)


def _mlp2(h1, stats, g1r, be1r, W2, b2r, g2r, be2r, w3r, b3r):
    return pl.pallas_call(
        _mlp2_body,
        out_shape=jax.ShapeDtypeStruct((_B, 1), jnp.float32),
    )(h1, stats, g1r, be1r, W2, b2r, g2r, be2r, w3r, b3r)


def kernel(cat_data, num_data, emb_tables, W1, b1, g1, be1, W2, b2, g2, be2, W3, b3):
    table16 = emb_tables.reshape(_TW, 16)
    offs = (jnp.arange(_F, dtype=jnp.int32) * _V)[None, :]
    flat_idx = (cat_data + offs).reshape(_BF)
    w0 = (flat_idx * 25) >> 3
    idx4 = (w0[:, None] + jnp.arange(4, dtype=jnp.int32)[None, :]).reshape(
        _NW * 4 * _NCH, _CH)
    shift = ((flat_idx & 7) << 1).reshape(_NW * _NCH, _CH)

    x_flat = _sc_gather()(table16, idx4, shift)
    x = x_flat.reshape(_B, _F * _D)

    nump = jnp.pad(num_data, ((0, 0), (0, 3)))
    w1a = W1[:_F * _D]
    w1b = jnp.pad(W1[_F * _D:], ((0, 3), (0, 0)))

    h1, stats = _mlp1(x, nump, w1a, w1b, b1.reshape(1, -1))
    out = _mlp2(h1, stats, g1.reshape(1, -1), be1.reshape(1, -1),
                W2, b2.reshape(1, -1), g2.reshape(1, -1), be2.reshape(1, -1),
                W3.reshape(1, -1), b3.reshape(1, 1))
    return out.reshape(_B)


# speedup vs baseline: 6.8750x; 1.3996x over previous
import functools

import jax
import jax.numpy as jnp
from jax import lax
from jax.experimental import pallas as pl
from jax.experimental.pallas import tpu as pltpu
from jax.experimental.pallas import tpu_sc as plsc

_B, _F, _V, _D = 16384, 26, 100000, 50
_DP = 64
_EPS = 1e-5
_NC, _NS = 2, 16
_NW = _NC * _NS
_BF = _B * _F
_PER_W = _BF // _NW
_CH = 128
_NCH = _PER_W // _CH
_TW = _F * _V * 8


def _gather_body(table3d_hbm, idx4_hbm, out_hbm,
                 idx4_v, win_a, win_b, sem_a, sem_b, sem_oa, sem_ob):
    table_hbm = table3d_hbm
    wid = lax.axis_index("s") * _NC + lax.axis_index("c")
    pltpu.sync_copy(idx4_hbm.at[pl.ds(wid * 4 * _NCH, 4 * _NCH)], idx4_v)

    def fire(j, win, sem):
        for t in range(4):
            pltpu.async_copy(table_hbm.at[idx4_v.at[j * 4 + t]],
                             win.at[pl.ds(t * 128, 128)], sem)

    def drain(win, sem):
        for t in range(4):
            pltpu.make_async_copy(table_hbm.at[idx4_v.at[0]],
                                  win.at[pl.ds(t * 128, 128)], sem).wait()

    fire(0, win_a, sem_a)

    def body(j, _):
        even = lax.rem(j, 2) == 0

        def store_wait(win, sem_o):
            pltpu.make_async_copy(win, out_hbm.at[pl.ds(0, 512)], sem_o).wait()

        @pl.when(jnp.logical_and(j + 1 < _NCH, even))
        def _():
            @pl.when(j >= 1)
            def _():
                store_wait(win_b, sem_ob)
            fire(j + 1, win_b, sem_b)

        @pl.when(jnp.logical_and(j + 1 < _NCH, jnp.logical_not(even)))
        def _():
            store_wait(win_a, sem_oa)
            fire(j + 1, win_a, sem_a)

        def store(win, sem_o):
            pltpu.async_copy(win, out_hbm.at[pl.ds(wid * 4 * _PER_W + j * 512, 512)],
                             sem_o)

        @pl.when(even)
        def _():
            drain(win_a, sem_a)
            store(win_a, sem_oa)

        @pl.when(jnp.logical_not(even))
        def _():
            drain(win_b, sem_b)
            store(win_b, sem_ob)

        return 0

    lax.fori_loop(0, _NCH, body, 0)
    pltpu.make_async_copy(win_a, out_hbm.at[pl.ds(0, 512)], sem_oa).wait()
    pltpu.make_async_copy(win_b, out_hbm.at[pl.ds(0, 512)], sem_ob).wait()


@functools.cache
def _sc_gather():
    return pl.kernel(
        _gather_body,
        out_type=jax.ShapeDtypeStruct((_BF * 4, 16), jnp.float32),
        mesh=plsc.VectorSubcoreMesh(
            core_axis_name="c", subcore_axis_name="s",
            num_cores=_NC, num_subcores=_NS),
        scratch_types=[
            pltpu.VMEM((4 * _NCH, _CH), jnp.int32),
            pltpu.VMEM((512, 16), jnp.float32),
            pltpu.VMEM((512, 16), jnp.float32),
            pltpu.SemaphoreType.DMA,
            pltpu.SemaphoreType.DMA,
            pltpu.SemaphoreType.DMA,
            pltpu.SemaphoreType.DMA,
        ],
        compiler_params=pltpu.CompilerParams(
            use_tc_tiling_on_sc=False, needs_layout_passes=False),
    )


_BB = 512


def _mlp1_body(x_ref, num_ref, w1a_ref, w1b_ref, b1_ref, h1_ref, stats_ref, acc_ref):
    i = pl.program_id(0)

    @pl.when(i == 0)
    def _():
        acc_ref[...] = jnp.zeros_like(acc_ref)

    h = jnp.dot(x_ref[...], w1a_ref[...], preferred_element_type=jnp.float32)
    h = h + jnp.dot(num_ref[...], w1b_ref[...], preferred_element_type=jnp.float32)
    h = jnp.maximum(h + b1_ref[...], 0.0)
    h1_ref[...] = h
    acc_ref[0:1, :] += jnp.sum(h, axis=0, keepdims=True)
    acc_ref[1:2, :] += jnp.sum(h * h, axis=0, keepdims=True)

    @pl.when(i == pl.num_programs(0) - 1)
    def _():
        stats_ref[...] = acc_ref[...]


def _mlp2_body(h1_ref, stats_ref, g1_ref, be1_ref, w2_ref, b2_ref, g2_ref,
               be2_ref, w3r_ref, b3_ref, out_ref):
    s = stats_ref[...]
    m1 = s[0:1, :] * (1.0 / _B)
    v1 = s[1:2, :] * (1.0 / _B) - m1 * m1
    h1n = (h1_ref[...] - m1) * lax.rsqrt(v1 + _EPS) * g1_ref[...] + be1_ref[...]
    h2 = jnp.dot(h1n, w2_ref[...], preferred_element_type=jnp.float32)
    h2 = jnp.maximum(h2 + b2_ref[...], 0.0)
    m2 = jnp.mean(h2, axis=0, keepdims=True)
    v2 = jnp.mean(h2 * h2, axis=0, keepdims=True) - m2 * m2
    h2n = (h2 - m2) * lax.rsqrt(v2 + _EPS) * g2_ref[...] + be2_ref[...]
    out_ref[...] = jnp.sum(h2n * w3r_ref[...], axis=1, keepdims=True) + b3_ref[...]


def _mlp1(x, nump, w1e, w1b, b1r):
    grid = (_B // _BB,)
    return pl.pallas_call(
        _mlp1_body,
        grid=grid,
        in_specs=[
            pl.BlockSpec((_BB, _F * _DP), lambda i: (i, 0)),
            pl.BlockSpec((_BB, 16), lambda i: (i, 0)),
            pl.BlockSpec((_F * _DP, 128), lambda i: (0, 0)),
            pl.BlockSpec((16, 128), lambda i: (0, 0)),
            pl.BlockSpec((1, 128), lambda i: (0, 0)),
        ],
        out_specs=[
            pl.BlockSpec((_BB, 128), lambda i: (i, 0)),
            pl.BlockSpec((8, 128), lambda i: (0, 0)),
        ],
        out_shape=[
            jax.ShapeDtypeStruct((_B, 128), jnp.float32),
            jax.ShapeDtypeStruct((8, 128), jnp.float32),
        ],
        scratch_shapes=[pltpu.VMEM((8, 128), jnp.float32)],
    )(x, nump, w1e, w1b, b1r)


def _mlp2(h1, stats, g1r, be1r, W2, b2r, g2r, be2r, w3r, b3r):
    return pl.pallas_call(
        _mlp2_body,
        out_shape=jax.ShapeDtypeStruct((_B, 1), jnp.float32),
    )(h1, stats, g1r, be1r, W2, b2r, g2r, be2r, w3r, b3r)


def kernel(cat_data, num_data, emb_tables, W1, b1, g1, be1, W2, b2, g2, be2, W3, b3):
    t16 = jnp.pad(emb_tables, ((0, 0), (0, 0), (0, 128 - _D))).reshape(_TW, 16)

    offs = (jnp.arange(_F, dtype=jnp.int32) * _V)[None, :]
    flat_idx = (cat_data + offs).reshape(_BF)
    w0 = flat_idx * 8
    idx4 = (w0[:, None] + jnp.arange(4, dtype=jnp.int32)[None, :]).reshape(
        _NW * 4 * _NCH, _CH)

    x = _sc_gather()(t16, idx4).reshape(_B, _F * _DP)

    nump = jnp.pad(num_data, ((0, 0), (0, 3)))
    w1e = jnp.pad(W1[:_F * _D].reshape(_F, _D, 128),
                  ((0, 0), (0, _DP - _D), (0, 0))).reshape(_F * _DP, 128)
    w1b = jnp.pad(W1[_F * _D:], ((0, 3), (0, 0)))

    h1, stats = _mlp1(x, nump, w1e, w1b, b1.reshape(1, -1))
    out = _mlp2(h1, stats, g1.reshape(1, -1), be1.reshape(1, -1),
                W2, b2.reshape(1, -1), g2.reshape(1, -1), be2.reshape(1, -1),
                W3.reshape(1, -1), b3.reshape(1, 1))
    return out.reshape(_B)
